# Initial kernel scaffold; baseline (speedup 1.0000x reference)
import functools

import jax
import jax.numpy as jnp
from jax import lax
from jax.experimental import pallas as pl
from jax.experimental.pallas import tpu as pltpu
from jax.experimental.pallas import tpu_sc as plsc

N = 10000
E = 160000
D = 256
DH = 128
R = 8

NT = 16
EPT = E // NT
CK = 80
NCH = EPT // CK
ROWS_PT = N // NT
CNT_SZ = 81920
CNT_PT = CNT_SZ // NT

BM = 1000


def _h_body(x_ref, w_ref, h_ref):
    h_ref[0] = jnp.dot(x_ref[...], w_ref[0], preferred_element_type=jnp.float32)


_h_call = pl.pallas_call(
    _h_body,
    grid=(N // BM, R),
    in_specs=[
        pl.BlockSpec((BM, D), lambda i, r: (i, 0)),
        pl.BlockSpec((1, D, D), lambda i, r: (r, 0, 0)),
    ],
    out_specs=pl.BlockSpec((1, BM, D), lambda i, r: (r, i, 0)),
    out_shape=jax.ShapeDtypeStruct((R, N, D), jnp.float32),
)


def _comb_body(x_ref, root_ref, bias_ref, s_ref, o_ref):
    o_ref[...] = (
        jnp.dot(x_ref[...], root_ref[...], preferred_element_type=jnp.float32)
        + bias_ref[...]
        + jnp.concatenate([s_ref[0], s_ref[1]], axis=-1)
    )


_comb_call = pl.pallas_call(
    _comb_body,
    grid=(N // BM,),
    in_specs=[
        pl.BlockSpec((BM, D), lambda i: (i, 0)),
        pl.BlockSpec((D, D), lambda i: (0, 0)),
        pl.BlockSpec((1, D), lambda i: (0, 0)),
        pl.BlockSpec((2, BM, DH), lambda i: (0, i, 0)),
    ],
    out_specs=pl.BlockSpec((BM, D), lambda i: (i, 0)),
    out_shape=jax.ShapeDtypeStruct((N, D), jnp.float32),
)


def _sc_body(et_hbm, src_hbm, dst_hbm, h2_hbm, out_hbm,
             et_all, src_all, dst_all, cidx, gidx, didx,
             ones, cbuf, wbuf, msg, zrows, zcnt, counts, acc, sem):
    c = lax.axis_index("c")
    s = lax.axis_index("s")

    zero16 = jnp.zeros((16,), jnp.float32)

    def _zr(i, _):
        zrows[i // 8, pl.ds((i % 8) * 16, 16)] = zero16
        return 0

    lax.fori_loop(0, 125 * 8, _zr, 0)

    def _zc(i, _):
        zcnt[pl.ds(i * 16, 16)] = zero16
        return 0

    lax.fori_loop(0, CNT_PT // 16, _zc, 0)

    for v in range(CK // 16):
        ones[pl.ds(v * 16, 16)] = jnp.ones((16,), jnp.float32)

    pltpu.sync_copy(zcnt, counts.at[pl.ds(s * CNT_PT, CNT_PT)])
    for k in range(ROWS_PT // 125):
        pltpu.sync_copy(zrows, acc.at[pl.ds(s * ROWS_PT + k * 125, 125)])

    base = s * EPT
    pltpu.sync_copy(et_hbm.at[pl.ds(base, EPT)], et_all)
    pltpu.sync_copy(src_hbm.at[pl.ds(base, EPT)], src_all)
    pltpu.sync_copy(dst_hbm.at[pl.ds(base, EPT)], dst_all)

    plsc.subcore_barrier()

    def _cnt(j, _):
        off = j * CK
        for v in range(CK // 16):
            sl = pl.ds(off + v * 16, 16)
            cidx[pl.ds(v * 16, 16)] = et_all[sl] * N + dst_all[sl]
        pltpu.sync_copy(ones, counts.at[cidx], add=True)
        return 0

    lax.fori_loop(0, NCH, _cnt, 0)

    plsc.subcore_barrier()

    def _main(j, _):
        off = j * CK
        for v in range(CK // 16):
            sl = pl.ds(off + v * 16, 16)
            e = et_all[sl]
            cidx[pl.ds(v * 16, 16)] = e * N + dst_all[sl]
            gidx[pl.ds(v * 16, 16)] = (e * N + src_all[sl]) * 2 + c
            didx[pl.ds(v * 16, 16)] = dst_all[sl]
        pltpu.async_copy(counts.at[cidx], cbuf, sem).wait()
        for v in range(CK // 16):
            sl = pl.ds(v * 16, 16)
            wbuf[sl] = 1.0 / jnp.maximum(cbuf[sl], 1.0)
        pltpu.async_copy(h2_hbm.at[gidx], msg, sem).wait()

        def _scale(i, _):
            w = wbuf[i]
            for q in range(DH // 16):
                msg[i, pl.ds(q * 16, 16)] = msg[i, pl.ds(q * 16, 16)] * w
            return 0

        lax.fori_loop(0, CK, _scale, 0)
        pltpu.sync_copy(msg, acc.at[didx], add=True)
        return 0

    lax.fori_loop(0, NCH, _main, 0)

    plsc.subcore_barrier()

    row0 = s * ROWS_PT
    pltpu.sync_copy(acc.at[pl.ds(row0, ROWS_PT)],
                    out_hbm.at[c, pl.ds(row0, ROWS_PT)])


_sc_call = functools.partial(
    pl.kernel,
    out_type=jax.ShapeDtypeStruct((2, N, DH), jnp.float32),
    mesh=plsc.VectorSubcoreMesh(core_axis_name="c", subcore_axis_name="s"),
    scratch_types=[
        pltpu.VMEM((EPT,), jnp.int32),
        pltpu.VMEM((EPT,), jnp.int32),
        pltpu.VMEM((EPT,), jnp.int32),
        pltpu.VMEM((CK,), jnp.int32),
        pltpu.VMEM((CK,), jnp.int32),
        pltpu.VMEM((CK,), jnp.int32),
        pltpu.VMEM((CK,), jnp.float32),
        pltpu.VMEM((CK,), jnp.float32),
        pltpu.VMEM((CK,), jnp.float32),
        pltpu.VMEM((CK, DH), jnp.float32),
        pltpu.VMEM((125, DH), jnp.float32),
        pltpu.VMEM((CNT_PT,), jnp.float32),
        pltpu.VMEM_SHARED((CNT_SZ,), jnp.float32),
        pltpu.VMEM_SHARED((N, DH), jnp.float32),
        pltpu.SemaphoreType.DMA,
    ],
)(_sc_body)


@jax.jit
def kernel(x, edge_index, edge_type, W, root, bias):
    src = edge_index[0].astype(jnp.int32)
    dst = edge_index[1].astype(jnp.int32)
    et = edge_type.astype(jnp.int32)
    h = _h_call(x, W)
    h2 = h.reshape(R * N * 2, DH)
    halves = _sc_call(et, src, dst, h2)
    return _comb_call(x, root, bias.reshape(1, D), halves)


# speedup vs baseline: 15.1491x; 15.1491x over previous
import functools

import jax
import jax.numpy as jnp
from jax import lax
from jax.experimental import pallas as pl
from jax.experimental.pallas import tpu as pltpu
from jax.experimental.pallas import tpu_sc as plsc

N = 10000
E = 160000
D = 256
DH = 128
R = 8

NT = 16
EPT = E // NT
CK = 80
NCH = EPT // CK
ACC_N = 10240
ROWS_PT = ACC_N // NT
CNT_SZ = 81920
CNT_PT = CNT_SZ // NT

BM = 1000


def _h_body(x_ref, w_ref, h_ref):
    h_ref[0] = jnp.dot(x_ref[...], w_ref[0], preferred_element_type=jnp.float32)


_h_call = pl.pallas_call(
    _h_body,
    grid=(N // BM, R),
    in_specs=[
        pl.BlockSpec((BM, D), lambda i, r: (i, 0)),
        pl.BlockSpec((1, D, D), lambda i, r: (r, 0, 0)),
    ],
    out_specs=pl.BlockSpec((1, BM, D), lambda i, r: (r, i, 0)),
    out_shape=jax.ShapeDtypeStruct((R, N, D), jnp.float32),
)


def _comb_body(x_ref, root_ref, bias_ref, s_ref, o_ref):
    o_ref[...] = (
        jnp.dot(x_ref[...], root_ref[...], preferred_element_type=jnp.float32)
        + bias_ref[...]
        + jnp.concatenate([s_ref[0], s_ref[1]], axis=-1)
    )


_comb_call = pl.pallas_call(
    _comb_body,
    grid=(N // BM,),
    in_specs=[
        pl.BlockSpec((BM, D), lambda i: (i, 0)),
        pl.BlockSpec((D, D), lambda i: (0, 0)),
        pl.BlockSpec((1, D), lambda i: (0, 0)),
        pl.BlockSpec((2, BM, DH), lambda i: (0, i, 0)),
    ],
    out_specs=pl.BlockSpec((BM, D), lambda i: (i, 0)),
    out_shape=jax.ShapeDtypeStruct((N, D), jnp.float32),
)


def _sc_body(pk_hbm, h2_hbm, out_hbm,
             pk_all, cidx, gidx, didx,
             ones, cbuf, wbuf, msg, zrows, zcnt, counts, acc, sem):
    c = lax.axis_index("c")
    s = lax.axis_index("s")

    zero16 = jnp.zeros((16,), jnp.float32)

    def _zr(i, _):
        zrows[i // 8, pl.ds((i % 8) * 16, 16)] = zero16
        return 0

    lax.fori_loop(0, 128 * 8, _zr, 0)

    def _zc(i, _):
        zcnt[pl.ds(i * 16, 16)] = zero16
        return 0

    lax.fori_loop(0, CNT_PT // 16, _zc, 0)

    for v in range(CK // 16):
        ones[pl.ds(v * 16, 16)] = jnp.ones((16,), jnp.float32)

    pltpu.sync_copy(zcnt, counts.at[pl.ds(s * CNT_PT, CNT_PT)])
    for k in range(ROWS_PT // 128):
        pltpu.sync_copy(zrows, acc.at[pl.ds(s * ROWS_PT + k * 128, 128)])

    base = s * EPT
    pltpu.sync_copy(pk_hbm.at[pl.ds(base, EPT)], pk_all)

    plsc.subcore_barrier()

    def _cnt(j, _):
        off = j * CK
        for v in range(CK // 16):
            pkv = pk_all[pl.ds(off + v * 16, 16)]
            d = pkv & 16383
            e = lax.shift_right_logical(pkv, 28)
            cidx[pl.ds(v * 16, 16)] = e * N + d
        pltpu.sync_copy(ones, counts.at[cidx], add=True)
        return 0

    lax.fori_loop(0, NCH, _cnt, 0)

    plsc.subcore_barrier()

    def _main(j, _):
        off = j * CK
        for v in range(CK // 16):
            pkv = pk_all[pl.ds(off + v * 16, 16)]
            d = pkv & 16383
            sr = lax.shift_right_logical(pkv, 14) & 16383
            e = lax.shift_right_logical(pkv, 28)
            en = e * N
            cidx[pl.ds(v * 16, 16)] = en + d
            gidx[pl.ds(v * 16, 16)] = (en + sr) * 2 + c
            didx[pl.ds(v * 16, 16)] = d
        pltpu.async_copy(counts.at[cidx], cbuf, sem).wait()
        for v in range(CK // 16):
            sl = pl.ds(v * 16, 16)
            wbuf[sl] = 1.0 / jnp.maximum(cbuf[sl], 1.0)
        pltpu.async_copy(h2_hbm.at[gidx], msg, sem).wait()

        def _scale(g, _):
            wv = wbuf[pl.ds(g * 16, 16)]
            for l in range(16):
                w = wv[l]
                row = g * 16 + l
                for q in range(DH // 16):
                    msg[row, pl.ds(q * 16, 16)] = msg[row, pl.ds(q * 16, 16)] * w
            return 0

        lax.fori_loop(0, CK // 16, _scale, 0)
        pltpu.sync_copy(msg, acc.at[didx], add=True)
        return 0

    lax.fori_loop(0, NCH, _main, 0)

    plsc.subcore_barrier()

    row0 = s * ROWS_PT
    pltpu.sync_copy(acc.at[pl.ds(row0, ROWS_PT)],
                    out_hbm.at[c, pl.ds(row0, ROWS_PT)])


_sc_call = functools.partial(
    pl.kernel,
    out_type=jax.ShapeDtypeStruct((2, ACC_N, DH), jnp.float32),
    mesh=plsc.VectorSubcoreMesh(core_axis_name="c", subcore_axis_name="s"),
    scratch_types=[
        pltpu.VMEM((EPT,), jnp.int32),
        pltpu.VMEM((CK,), jnp.int32),
        pltpu.VMEM((CK,), jnp.int32),
        pltpu.VMEM((CK,), jnp.int32),
        pltpu.VMEM((CK,), jnp.float32),
        pltpu.VMEM((CK,), jnp.float32),
        pltpu.VMEM((CK,), jnp.float32),
        pltpu.VMEM((CK, DH), jnp.float32),
        pltpu.VMEM((128, DH), jnp.float32),
        pltpu.VMEM((CNT_PT,), jnp.float32),
        pltpu.VMEM_SHARED((CNT_SZ,), jnp.float32),
        pltpu.VMEM_SHARED((ACC_N, DH), jnp.float32),
        pltpu.SemaphoreType.DMA,
    ],
)(_sc_body)


@jax.jit
def kernel(x, edge_index, edge_type, W, root, bias):
    src = edge_index[0].astype(jnp.int32)
    dst = edge_index[1].astype(jnp.int32)
    et = edge_type.astype(jnp.int32)
    pk = (et << 28) | (src << 14) | dst
    h = _h_call(x, W)
    h2 = h.reshape(R * N * 2, DH)
    halves = _sc_call(pk, h2)
    return _comb_call(x, root, bias.reshape(1, D), halves)


# speedup vs baseline: 20.7641x; 1.3706x over previous
import functools

import jax
import jax.numpy as jnp
from jax import lax
from jax.experimental import pallas as pl
from jax.experimental.pallas import tpu as pltpu
from jax.experimental.pallas import tpu_sc as plsc

N = 10000
E = 160000
D = 256
DH = 128
R = 8

NT = 16
EPT = E // NT
CK = 80
NCH = EPT // CK
ACC_N = 10240
ROWS_PT = ACC_N // NT
CNT_SZ = 81920
CNT_PT = CNT_SZ // NT

BM = 1000


def _h_body(x_ref, w_ref, h_ref):
    h_ref[0] = jnp.dot(x_ref[...], w_ref[0], preferred_element_type=jnp.float32)


_h_call = pl.pallas_call(
    _h_body,
    grid=(N // BM, R),
    in_specs=[
        pl.BlockSpec((BM, D), lambda i, r: (i, 0)),
        pl.BlockSpec((1, D, D), lambda i, r: (r, 0, 0)),
    ],
    out_specs=pl.BlockSpec((1, BM, D), lambda i, r: (r, i, 0)),
    out_shape=jax.ShapeDtypeStruct((R, N, D), jnp.float32),
)


def _comb_body(x_ref, root_ref, bias_ref, s_ref, o_ref):
    o_ref[...] = (
        jnp.dot(x_ref[...], root_ref[...], preferred_element_type=jnp.float32)
        + bias_ref[...]
        + jnp.concatenate([s_ref[0], s_ref[1]], axis=-1)
    )


_comb_call = pl.pallas_call(
    _comb_body,
    grid=(N // BM,),
    in_specs=[
        pl.BlockSpec((BM, D), lambda i: (i, 0)),
        pl.BlockSpec((D, D), lambda i: (0, 0)),
        pl.BlockSpec((1, D), lambda i: (0, 0)),
        pl.BlockSpec((2, BM, DH), lambda i: (0, i, 0)),
    ],
    out_specs=pl.BlockSpec((BM, D), lambda i: (i, 0)),
    out_shape=jax.ShapeDtypeStruct((N, D), jnp.float32),
)


def _sc_body(pk_hbm, h2_hbm, out_hbm,
             pk_all, cidxa, cidxb, gidx0, gidx1, didx0, didx1,
             ones, cbuf, wbuf0, wbuf1, msg0, msg1, zrows, zcnt, counts, acc,
             semg0, semg1, sems0, sems1, semc, semcnt):
    c = lax.axis_index("c")
    s = lax.axis_index("s")

    zero16 = jnp.zeros((16,), jnp.float32)

    def _zr(i, _):
        zrows[i // 8, pl.ds((i % 8) * 16, 16)] = zero16
        return 0

    lax.fori_loop(0, 16 * 8, _zr, 0)

    def _zc(i, _):
        zcnt[pl.ds(i * 16, 16)] = zero16
        return 0

    lax.fori_loop(0, 1280 // 16, _zc, 0)

    for v in range(CK // 16):
        ones[pl.ds(v * 16, 16)] = jnp.ones((16,), jnp.float32)

    for k in range(CNT_PT // 1280):
        pltpu.sync_copy(zcnt, counts.at[pl.ds(s * CNT_PT + k * 1280, 1280)])
    for k in range(ROWS_PT // 16):
        pltpu.sync_copy(zrows, acc.at[pl.ds(s * ROWS_PT + k * 16, 16)])

    base = s * EPT
    pltpu.sync_copy(pk_hbm.at[pl.ds(base, EPT)], pk_all)

    plsc.subcore_barrier()

    def _cidx(j, ci):
        off = j * CK
        for v in range(CK // 16):
            pkv = pk_all[pl.ds(off + v * 16, 16)]
            d = pkv & 16383
            e = lax.shift_right_logical(pkv, 28)
            ci[pl.ds(v * 16, 16)] = e * N + d

    def _cnt_fire(ci):
        pltpu.async_copy(ones, counts.at[ci], semcnt, add=True)

    def _cnt_wait():
        pltpu.make_async_copy(ones, counts.at[cidxa], semcnt).wait()

    _cidx(0, cidxa)
    _cnt_fire(cidxa)
    _cidx(1, cidxb)
    _cnt_fire(cidxb)

    def _cnt(t, _):
        j = 2 * t + 2
        _cnt_wait()
        _cidx(j, cidxa)
        _cnt_fire(cidxa)
        _cnt_wait()
        _cidx(j + 1, cidxb)
        _cnt_fire(cidxb)
        return 0

    lax.fori_loop(0, (NCH - 3) // 2, _cnt, 0)
    _cnt_wait()
    _cidx(NCH - 1, cidxa)
    _cnt_fire(cidxa)
    _cnt_wait()
    _cnt_wait()

    plsc.subcore_barrier()

    def _idx(j, gi, di, wb):
        off = j * CK
        for v in range(CK // 16):
            pkv = pk_all[pl.ds(off + v * 16, 16)]
            d = pkv & 16383
            sr = lax.shift_right_logical(pkv, 14) & 16383
            e = lax.shift_right_logical(pkv, 28)
            cidxa[pl.ds(v * 16, 16)] = e * N + d
            gi[pl.ds(v * 16, 16)] = (e * N + sr) * 2 + c
            di[pl.ds(v * 16, 16)] = d
        pltpu.async_copy(counts.at[cidxa], cbuf, semc).wait()
        for v in range(CK // 16):
            sl = pl.ds(v * 16, 16)
            wb[sl] = 1.0 / jnp.maximum(cbuf[sl], 1.0)

    def _scale(mg, wb):
        def _rows(g, _):
            wv = wb[pl.ds(g * 16, 16)]
            for l in range(16):
                w = wv[l]
                row = g * 16 + l
                for q in range(DH // 16):
                    mg[row, pl.ds(q * 16, 16)] = mg[row, pl.ds(q * 16, 16)] * w
            return 0

        lax.fori_loop(0, CK // 16, _rows, 0)

    def _g_start(gi, mg, sem):
        pltpu.async_copy(h2_hbm.at[gi], mg, sem)

    def _g_wait(gi, mg, sem):
        pltpu.make_async_copy(h2_hbm.at[gi], mg, sem).wait()

    def _s_start(mg, di, sem):
        pltpu.async_copy(mg, acc.at[di], sem, add=True)

    def _s_wait(mg, di, sem):
        pltpu.make_async_copy(mg, acc.at[di], sem).wait()

    _idx(0, gidx0, didx0, wbuf0)
    _g_start(gidx0, msg0, semg0)
    _idx(1, gidx1, didx1, wbuf1)
    _g_start(gidx1, msg1, semg1)
    _g_wait(gidx0, msg0, semg0)
    _scale(msg0, wbuf0)
    _s_start(msg0, didx0, sems0)

    def _pipe(t, _):
        j = 2 * t + 2
        _s_wait(msg0, didx0, sems0)
        _idx(j, gidx0, didx0, wbuf0)
        _g_start(gidx0, msg0, semg0)
        _g_wait(gidx1, msg1, semg1)
        _scale(msg1, wbuf1)
        _s_start(msg1, didx1, sems1)
        _s_wait(msg1, didx1, sems1)
        _idx(j + 1, gidx1, didx1, wbuf1)
        _g_start(gidx1, msg1, semg1)
        _g_wait(gidx0, msg0, semg0)
        _scale(msg0, wbuf0)
        _s_start(msg0, didx0, sems0)
        return 0

    lax.fori_loop(0, (NCH - 3) // 2, _pipe, 0)

    _s_wait(msg0, didx0, sems0)
    _idx(NCH - 1, gidx0, didx0, wbuf0)
    _g_start(gidx0, msg0, semg0)
    _g_wait(gidx1, msg1, semg1)
    _scale(msg1, wbuf1)
    _s_start(msg1, didx1, sems1)
    _g_wait(gidx0, msg0, semg0)
    _scale(msg0, wbuf0)
    _s_start(msg0, didx0, sems0)
    _s_wait(msg1, didx1, sems1)
    _s_wait(msg0, didx0, sems0)

    plsc.subcore_barrier()

    row0 = s * ROWS_PT
    pltpu.sync_copy(acc.at[pl.ds(row0, ROWS_PT)],
                    out_hbm.at[c, pl.ds(row0, ROWS_PT)])


_sc_call = functools.partial(
    pl.kernel,
    out_type=jax.ShapeDtypeStruct((2, ACC_N, DH), jnp.float32),
    mesh=plsc.VectorSubcoreMesh(core_axis_name="c", subcore_axis_name="s"),
    scratch_types=[
        pltpu.VMEM((EPT,), jnp.int32),
        pltpu.VMEM((CK,), jnp.int32),
        pltpu.VMEM((CK,), jnp.int32),
        pltpu.VMEM((CK,), jnp.int32),
        pltpu.VMEM((CK,), jnp.int32),
        pltpu.VMEM((CK,), jnp.int32),
        pltpu.VMEM((CK,), jnp.int32),
        pltpu.VMEM((CK,), jnp.float32),
        pltpu.VMEM((CK,), jnp.float32),
        pltpu.VMEM((CK,), jnp.float32),
        pltpu.VMEM((CK,), jnp.float32),
        pltpu.VMEM((CK, DH), jnp.float32),
        pltpu.VMEM((CK, DH), jnp.float32),
        pltpu.VMEM((16, DH), jnp.float32),
        pltpu.VMEM((1280,), jnp.float32),
        pltpu.VMEM_SHARED((CNT_SZ,), jnp.float32),
        pltpu.VMEM_SHARED((ACC_N, DH), jnp.float32),
        pltpu.SemaphoreType.DMA,
        pltpu.SemaphoreType.DMA,
        pltpu.SemaphoreType.DMA,
        pltpu.SemaphoreType.DMA,
        pltpu.SemaphoreType.DMA,
        pltpu.SemaphoreType.DMA,
    ],
)(_sc_body)


@jax.jit
def kernel(x, edge_index, edge_type, W, root, bias):
    src = edge_index[0].astype(jnp.int32)
    dst = edge_index[1].astype(jnp.int32)
    et = edge_type.astype(jnp.int32)
    pk = (et << 28) | (src << 14) | dst
    h = _h_call(x, W)
    h2 = h.reshape(R * N * 2, DH)
    halves = _sc_call(pk, h2)
    return _comb_call(x, root, bias.reshape(1, D), halves)


# speedup vs baseline: 26.6864x; 1.2852x over previous
import functools

import jax
import jax.numpy as jnp
from jax import lax
from jax.experimental import pallas as pl
from jax.experimental.pallas import tpu as pltpu
from jax.experimental.pallas import tpu_sc as plsc

N = 10000
E = 160000
D = 256
DH = 128
R = 8

NT = 16
EPT = E // NT
CK = 80
NCH = EPT // CK
ACC_N = 10240
ROWS_PT = ACC_N // NT
CNT_SZ = 81920
CNT_PT = CNT_SZ // NT

BM = 1000


def _h_body(x_ref, w_ref, h_ref):
    res = jnp.dot(x_ref[...], w_ref[0], preferred_element_type=jnp.float32)
    h_ref[0, 0] = res[:, :DH]
    h_ref[1, 0] = res[:, DH:]


_h_call = pl.pallas_call(
    _h_body,
    grid=(N // BM, R),
    in_specs=[
        pl.BlockSpec((BM, D), lambda i, r: (i, 0)),
        pl.BlockSpec((1, D, D), lambda i, r: (r, 0, 0)),
    ],
    out_specs=pl.BlockSpec((2, 1, BM, DH), lambda i, r: (0, r, i, 0)),
    out_shape=jax.ShapeDtypeStruct((2, R, N, DH), jnp.float32),
)


def _comb_body(x_ref, root_ref, bias_ref, s_ref, o_ref):
    o_ref[...] = (
        jnp.dot(x_ref[...], root_ref[...], preferred_element_type=jnp.float32)
        + bias_ref[...]
        + jnp.concatenate([s_ref[0], s_ref[1]], axis=-1)
    )


_comb_call = pl.pallas_call(
    _comb_body,
    grid=(N // BM,),
    in_specs=[
        pl.BlockSpec((BM, D), lambda i: (i, 0)),
        pl.BlockSpec((D, D), lambda i: (0, 0)),
        pl.BlockSpec((1, D), lambda i: (0, 0)),
        pl.BlockSpec((2, BM, DH), lambda i: (0, i, 0)),
    ],
    out_specs=pl.BlockSpec((BM, D), lambda i: (i, 0)),
    out_shape=jax.ShapeDtypeStruct((N, D), jnp.float32),
)


def _sc_body(pk_hbm, h2_hbm, out_hbm,
             pk_all, cidxa, cidxb, gidx0, gidx1, didx0, didx1,
             ones, cbuf, wbuf0, wbuf1, msg0, msg1, zrows, zcnt, counts, acc,
             semg0, semg1, sems0, sems1, semc, semcnt):
    c = lax.axis_index("c")
    s = lax.axis_index("s")

    zero16 = jnp.zeros((16,), jnp.float32)

    def _zr(i, _):
        zrows[i // 8, pl.ds((i % 8) * 16, 16)] = zero16
        return 0

    lax.fori_loop(0, 16 * 8, _zr, 0)

    def _zc(i, _):
        zcnt[pl.ds(i * 16, 16)] = zero16
        return 0

    lax.fori_loop(0, 1280 // 16, _zc, 0)

    for v in range(CK // 16):
        ones[pl.ds(v * 16, 16)] = jnp.ones((16,), jnp.float32)

    for k in range(CNT_PT // 1280):
        pltpu.sync_copy(zcnt, counts.at[pl.ds(s * CNT_PT + k * 1280, 1280)])
    for k in range(ROWS_PT // 16):
        pltpu.sync_copy(zrows, acc.at[pl.ds(s * ROWS_PT + k * 16, 16)])

    base = s * EPT
    pltpu.sync_copy(pk_hbm.at[pl.ds(base, EPT)], pk_all)

    plsc.subcore_barrier()

    def _cidx(j, ci):
        off = j * CK
        for v in range(CK // 16):
            pkv = pk_all[pl.ds(off + v * 16, 16)]
            d = pkv & 16383
            e = lax.shift_right_logical(pkv, 28)
            ci[pl.ds(v * 16, 16)] = e * N + d

    def _cnt_fire(ci):
        pltpu.async_copy(ones, counts.at[ci], semcnt, add=True)

    def _cnt_wait():
        pltpu.make_async_copy(ones, counts.at[cidxa], semcnt).wait()

    _cidx(0, cidxa)
    _cnt_fire(cidxa)
    _cidx(1, cidxb)
    _cnt_fire(cidxb)

    def _cnt(t, _):
        j = 2 * t + 2
        _cnt_wait()
        _cidx(j, cidxa)
        _cnt_fire(cidxa)
        _cnt_wait()
        _cidx(j + 1, cidxb)
        _cnt_fire(cidxb)
        return 0

    lax.fori_loop(0, (NCH - 3) // 2, _cnt, 0)
    _cnt_wait()
    _cidx(NCH - 1, cidxa)
    _cnt_fire(cidxa)
    _cnt_wait()
    _cnt_wait()

    plsc.subcore_barrier()

    def _idx(j, gi, di, wb):
        off = j * CK
        for v in range(CK // 16):
            pkv = pk_all[pl.ds(off + v * 16, 16)]
            d = pkv & 16383
            sr = lax.shift_right_logical(pkv, 14) & 16383
            e = lax.shift_right_logical(pkv, 28)
            cidxa[pl.ds(v * 16, 16)] = e * N + d
            gi[pl.ds(v * 16, 16)] = c * (R * N) + e * N + sr
            di[pl.ds(v * 16, 16)] = d
        pltpu.async_copy(counts.at[cidxa], cbuf, semc).wait()
        for v in range(CK // 16):
            sl = pl.ds(v * 16, 16)
            wb[sl] = 1.0 / jnp.maximum(cbuf[sl], 1.0)

    def _scale(mg, wb):
        def _rows(g, _):
            wv = wb[pl.ds(g * 16, 16)]
            for l in range(16):
                w = wv[l]
                row = g * 16 + l
                for q in range(DH // 16):
                    mg[row, pl.ds(q * 16, 16)] = mg[row, pl.ds(q * 16, 16)] * w
            return 0

        lax.fori_loop(0, CK // 16, _rows, 0)

    def _g_start(gi, mg, sem):
        pltpu.async_copy(h2_hbm.at[gi], mg, sem)

    def _g_wait(gi, mg, sem):
        pltpu.make_async_copy(h2_hbm.at[gi], mg, sem).wait()

    def _s_start(mg, di, sem):
        pltpu.async_copy(mg, acc.at[di], sem, add=True)

    def _s_wait(mg, di, sem):
        pltpu.make_async_copy(mg, acc.at[di], sem).wait()

    _idx(0, gidx0, didx0, wbuf0)
    _g_start(gidx0, msg0, semg0)
    _idx(1, gidx1, didx1, wbuf1)
    _g_start(gidx1, msg1, semg1)
    _g_wait(gidx0, msg0, semg0)
    _scale(msg0, wbuf0)
    _s_start(msg0, didx0, sems0)

    def _pipe(t, _):
        j = 2 * t + 2
        _s_wait(msg0, didx0, sems0)
        _idx(j, gidx0, didx0, wbuf0)
        _g_start(gidx0, msg0, semg0)
        _g_wait(gidx1, msg1, semg1)
        _scale(msg1, wbuf1)
        _s_start(msg1, didx1, sems1)
        _s_wait(msg1, didx1, sems1)
        _idx(j + 1, gidx1, didx1, wbuf1)
        _g_start(gidx1, msg1, semg1)
        _g_wait(gidx0, msg0, semg0)
        _scale(msg0, wbuf0)
        _s_start(msg0, didx0, sems0)
        return 0

    lax.fori_loop(0, (NCH - 3) // 2, _pipe, 0)

    _s_wait(msg0, didx0, sems0)
    _idx(NCH - 1, gidx0, didx0, wbuf0)
    _g_start(gidx0, msg0, semg0)
    _g_wait(gidx1, msg1, semg1)
    _scale(msg1, wbuf1)
    _s_start(msg1, didx1, sems1)
    _g_wait(gidx0, msg0, semg0)
    _scale(msg0, wbuf0)
    _s_start(msg0, didx0, sems0)
    _s_wait(msg1, didx1, sems1)
    _s_wait(msg0, didx0, sems0)

    plsc.subcore_barrier()

    row0 = s * ROWS_PT
    pltpu.sync_copy(acc.at[pl.ds(row0, ROWS_PT)],
                    out_hbm.at[c, pl.ds(row0, ROWS_PT)])


_sc_call = functools.partial(
    pl.kernel,
    out_type=jax.ShapeDtypeStruct((2, ACC_N, DH), jnp.float32),
    mesh=plsc.VectorSubcoreMesh(core_axis_name="c", subcore_axis_name="s"),
    scratch_types=[
        pltpu.VMEM((EPT,), jnp.int32),
        pltpu.VMEM((CK,), jnp.int32),
        pltpu.VMEM((CK,), jnp.int32),
        pltpu.VMEM((CK,), jnp.int32),
        pltpu.VMEM((CK,), jnp.int32),
        pltpu.VMEM((CK,), jnp.int32),
        pltpu.VMEM((CK,), jnp.int32),
        pltpu.VMEM((CK,), jnp.float32),
        pltpu.VMEM((CK,), jnp.float32),
        pltpu.VMEM((CK,), jnp.float32),
        pltpu.VMEM((CK,), jnp.float32),
        pltpu.VMEM((CK, DH), jnp.float32),
        pltpu.VMEM((CK, DH), jnp.float32),
        pltpu.VMEM((16, DH), jnp.float32),
        pltpu.VMEM((1280,), jnp.float32),
        pltpu.VMEM_SHARED((CNT_SZ,), jnp.float32),
        pltpu.VMEM_SHARED((ACC_N, DH), jnp.float32),
        pltpu.SemaphoreType.DMA,
        pltpu.SemaphoreType.DMA,
        pltpu.SemaphoreType.DMA,
        pltpu.SemaphoreType.DMA,
        pltpu.SemaphoreType.DMA,
        pltpu.SemaphoreType.DMA,
    ],
)(_sc_body)


@jax.jit
def kernel(x, edge_index, edge_type, W, root, bias):
    src = edge_index[0].astype(jnp.int32)
    dst = edge_index[1].astype(jnp.int32)
    et = edge_type.astype(jnp.int32)
    pk = (et << 28) | (src << 14) | dst
    h = _h_call(x, W)
    h2 = h.reshape(2 * R * N, DH)
    halves = _sc_call(pk, h2)
    return _comb_call(x, root, bias.reshape(1, D), halves)


# speedup vs baseline: 26.8045x; 1.0044x over previous
import functools

import jax
import jax.numpy as jnp
from jax import lax
from jax.experimental import pallas as pl
from jax.experimental.pallas import tpu as pltpu
from jax.experimental.pallas import tpu_sc as plsc

N = 10000
E = 160000
D = 256
DH = 128
R = 8

NT = 16
EPT = E // NT
CK = 80
NCH = EPT // CK
ACC_N = 10240
ROWS_PT = ACC_N // NT
CNT_SZ = 81920
CNT_PT = CNT_SZ // NT

BM = 1000

_PA = []
_PB = []
for _hf in range(2):
    for _q in range(4):
        for _i in range(16):
            _PA.append(_hf * 128 + 32 * _q + _i)
            _PB.append(_hf * 128 + 32 * _q + 16 + _i)


def _h_body(x_ref, w_ref, h_ref):
    res = jnp.dot(x_ref[...], w_ref[0], preferred_element_type=jnp.float32)
    a = res[:, :DH].astype(jnp.bfloat16).astype(jnp.float32)
    b = res[:, DH:].astype(jnp.bfloat16).astype(jnp.float32)
    ya = jax.lax.bitcast_convert_type(a, jnp.uint32)
    yb = jax.lax.bitcast_convert_type(b, jnp.uint32)
    packed = jax.lax.bitcast_convert_type(
        (ya >> 16) | (yb & jnp.uint32(0xFFFF0000)), jnp.float32)
    h_ref[0] = packed


_h_call = pl.pallas_call(
    _h_body,
    grid=(N // BM, R),
    in_specs=[
        pl.BlockSpec((BM, D), lambda i, r: (i, 0)),
        pl.BlockSpec((1, D, D), lambda i, r: (r, 0, 0)),
    ],
    out_specs=pl.BlockSpec((1, BM, DH), lambda i, r: (r, i, 0)),
    out_shape=jax.ShapeDtypeStruct((R, N, DH), jnp.float32),
)


def _comb_body(x_ref, root_ref, bias_ref, s_ref, o_ref):
    o_ref[...] = (
        jnp.dot(x_ref[...], root_ref[...], preferred_element_type=jnp.float32)
        + bias_ref[...]
        + jnp.concatenate([s_ref[0], s_ref[1]], axis=-1)
    )


_comb_call = pl.pallas_call(
    _comb_body,
    grid=(N // BM,),
    in_specs=[
        pl.BlockSpec((BM, D), lambda i: (i, 0)),
        pl.BlockSpec((D, D), lambda i: (0, 0)),
        pl.BlockSpec((1, D), lambda i: (0, 0)),
        pl.BlockSpec((2, BM, DH), lambda i: (0, i, 0)),
    ],
    out_specs=pl.BlockSpec((BM, D), lambda i: (i, 0)),
    out_shape=jax.ShapeDtypeStruct((N, D), jnp.float32),
)


def _sc_body(pk_hbm, h2_hbm, out_hbm,
             pk_all, cidxa, cidxb, gidx0, gidx1, didx0, didx1,
             ones, cbuf, wbuf0, wbuf1, mg0, mg1, zrows, zcnt, counts, acc,
             semg0, semg1, sems0, sems1, semc, semcnt):
    c = lax.axis_index("c")
    s = lax.axis_index("s")

    zero16 = jnp.zeros((16,), jnp.float32)

    def _zr(i, _):
        zrows[i // 8, pl.ds((i % 8) * 16, 16)] = zero16
        return 0

    lax.fori_loop(0, 16 * 8, _zr, 0)

    def _zc(i, _):
        zcnt[pl.ds(i * 16, 16)] = zero16
        return 0

    lax.fori_loop(0, 1280 // 16, _zc, 0)

    for v in range(CK // 16):
        ones[pl.ds(v * 16, 16)] = jnp.ones((16,), jnp.float32)

    for k in range(CNT_PT // 1280):
        pltpu.sync_copy(zcnt, counts.at[pl.ds(s * CNT_PT + k * 1280, 1280)])
    for k in range(ROWS_PT // 16):
        pltpu.sync_copy(zrows, acc.at[pl.ds(s * ROWS_PT + k * 16, 16)])

    base = s * EPT
    pltpu.sync_copy(pk_hbm.at[pl.ds(base, EPT)], pk_all)

    plsc.subcore_barrier()

    def _cidx(j, ci):
        off = j * CK
        for v in range(CK // 16):
            pkv = pk_all[pl.ds(off + v * 16, 16)]
            d = pkv & 16383
            e = lax.shift_right_logical(pkv, 28)
            ci[pl.ds(v * 16, 16)] = e * N + d

    def _cnt_fire(ci):
        pltpu.async_copy(ones, counts.at[ci], semcnt, add=True)

    def _cnt_wait():
        pltpu.make_async_copy(ones, counts.at[cidxa], semcnt).wait()

    _cidx(0, cidxa)
    _cnt_fire(cidxa)
    _cidx(1, cidxb)
    _cnt_fire(cidxb)

    def _cnt(t, _):
        j = 2 * t + 2
        _cnt_wait()
        _cidx(j, cidxa)
        _cnt_fire(cidxa)
        _cnt_wait()
        _cidx(j + 1, cidxb)
        _cnt_fire(cidxb)
        return 0

    lax.fori_loop(0, (NCH - 3) // 2, _cnt, 0)
    _cnt_wait()
    _cidx(NCH - 1, cidxa)
    _cnt_fire(cidxa)
    _cnt_wait()
    _cnt_wait()

    plsc.subcore_barrier()

    def _idx(j, gi, di, wb):
        off = j * CK
        for v in range(CK // 16):
            pkv = pk_all[pl.ds(off + v * 16, 16)]
            d = pkv & 16383
            sr = lax.shift_right_logical(pkv, 14) & 16383
            e = lax.shift_right_logical(pkv, 28)
            cidxa[pl.ds(v * 16, 16)] = e * N + d
            gi[pl.ds(v * 16, 16)] = e * N + sr
            di[pl.ds(v * 16, 16)] = d
        pltpu.async_copy(counts.at[cidxa], cbuf, semc).wait()
        for v in range(CK // 16):
            sl = pl.ds(v * 16, 16)
            wb[sl] = 1.0 / jnp.maximum(cbuf[sl], 1.0)

    _MASK_HI = jnp.int32(-65536)

    def _scale(mg, wb):
        def _unpack_q(mg, row, w, q, src_base):
            y = lax.bitcast_convert_type(
                mg[row, pl.ds(src_base + 16 * q, 16)], jnp.int32)
            lo = lax.bitcast_convert_type(lax.shift_left(y, 16), jnp.float32)
            hi = lax.bitcast_convert_type(y & _MASK_HI, jnp.float32)
            mg[row, pl.ds(32 * q, 16)] = lo * w
            mg[row, pl.ds(32 * q + 16, 16)] = hi * w

        def _rows(g, _):
            wv = wb[pl.ds(g * 16, 16)]
            for l in range(16):
                w = wv[l]
                row = g * 16 + l

                @pl.when(c == 0)
                def _():
                    for q in (3, 2, 1, 0):
                        _unpack_q(mg, row, w, q, 0)

                @pl.when(c != 0)
                def _():
                    for q in (0, 1, 2, 3):
                        _unpack_q(mg, row, w, q, 64)
            return 0

        lax.fori_loop(0, CK // 16, _rows, 0)

    def _g_start(gi, mg, sem):
        pltpu.async_copy(h2_hbm.at[gi], mg, sem)

    def _g_wait(gi, mg, sem):
        pltpu.make_async_copy(h2_hbm.at[gi], mg, sem).wait()

    def _s_start(mg, di, sem):
        pltpu.async_copy(mg, acc.at[di], sem, add=True)

    def _s_wait(mg, di, sem):
        pltpu.make_async_copy(mg, acc.at[di], sem).wait()

    _idx(0, gidx0, didx0, wbuf0)
    _g_start(gidx0, mg0, semg0)
    _idx(1, gidx1, didx1, wbuf1)
    _g_start(gidx1, mg1, semg1)
    _g_wait(gidx0, mg0, semg0)
    _scale(mg0, wbuf0)
    _s_start(mg0, didx0, sems0)

    def _pipe(t, _):
        j = 2 * t + 2
        _s_wait(mg0, didx0, sems0)
        _idx(j, gidx0, didx0, wbuf0)
        _g_start(gidx0, mg0, semg0)
        _g_wait(gidx1, mg1, semg1)
        _scale(mg1, wbuf1)
        _s_start(mg1, didx1, sems1)
        _s_wait(mg1, didx1, sems1)
        _idx(j + 1, gidx1, didx1, wbuf1)
        _g_start(gidx1, mg1, semg1)
        _g_wait(gidx0, mg0, semg0)
        _scale(mg0, wbuf0)
        _s_start(mg0, didx0, sems0)
        return 0

    lax.fori_loop(0, (NCH - 3) // 2, _pipe, 0)

    _s_wait(mg0, didx0, sems0)
    _idx(NCH - 1, gidx0, didx0, wbuf0)
    _g_start(gidx0, mg0, semg0)
    _g_wait(gidx1, mg1, semg1)
    _scale(mg1, wbuf1)
    _s_start(mg1, didx1, sems1)
    _g_wait(gidx0, mg0, semg0)
    _scale(mg0, wbuf0)
    _s_start(mg0, didx0, sems0)
    _s_wait(mg1, didx1, sems1)
    _s_wait(mg0, didx0, sems0)

    plsc.subcore_barrier()

    row0 = s * ROWS_PT
    pltpu.sync_copy(acc.at[pl.ds(row0, ROWS_PT)],
                    out_hbm.at[c, pl.ds(row0, ROWS_PT)])


_sc_call = functools.partial(
    pl.kernel,
    out_type=jax.ShapeDtypeStruct((2, ACC_N, DH), jnp.float32),
    mesh=plsc.VectorSubcoreMesh(core_axis_name="c", subcore_axis_name="s"),
    scratch_types=[
        pltpu.VMEM((EPT,), jnp.int32),
        pltpu.VMEM((CK,), jnp.int32),
        pltpu.VMEM((CK,), jnp.int32),
        pltpu.VMEM((CK,), jnp.int32),
        pltpu.VMEM((CK,), jnp.int32),
        pltpu.VMEM((CK,), jnp.int32),
        pltpu.VMEM((CK,), jnp.int32),
        pltpu.VMEM((CK,), jnp.float32),
        pltpu.VMEM((CK,), jnp.float32),
        pltpu.VMEM((CK,), jnp.float32),
        pltpu.VMEM((CK,), jnp.float32),
        pltpu.VMEM((CK, DH), jnp.float32),
        pltpu.VMEM((CK, DH), jnp.float32),
        pltpu.VMEM((16, DH), jnp.float32),
        pltpu.VMEM((1280,), jnp.float32),
        pltpu.VMEM_SHARED((CNT_SZ,), jnp.float32),
        pltpu.VMEM_SHARED((ACC_N, DH), jnp.float32),
        pltpu.SemaphoreType.DMA,
        pltpu.SemaphoreType.DMA,
        pltpu.SemaphoreType.DMA,
        pltpu.SemaphoreType.DMA,
        pltpu.SemaphoreType.DMA,
        pltpu.SemaphoreType.DMA,
    ],
)(_sc_body)


@jax.jit
def kernel(x, edge_index, edge_type, W, root, bias):
    W = jnp.concatenate(
        [W[:, :, jnp.array(_PA, dtype=jnp.int32)],
         W[:, :, jnp.array(_PB, dtype=jnp.int32)]], axis=2)
    src = edge_index[0].astype(jnp.int32)
    dst = edge_index[1].astype(jnp.int32)
    et = edge_type.astype(jnp.int32)
    pk = (et << 28) | (src << 14) | dst
    h = _h_call(x, W)
    h2 = h.reshape(R * N, DH)
    halves = _sc_call(pk, h2)
    return _comb_call(x, root, bias.reshape(1, D), halves)
